# baseline (speedup 1.0000x reference)
import functools

import jax
import jax.numpy as jnp
import numpy as np
from jax import lax
from jax.experimental import pallas as pl
from jax.experimental.pallas import tpu as pltpu
from jax.experimental.pallas import tpu_sc as plsc

N_LINES = 208
BATCH_SIZE = 128

_THRESHOLDS = np.array([
    0.4386325, 0.5337529, 0.44591832, 0.43839633, 0.8973628, 0.558946, 0.9838855,
    0.84199095, 0.38577962, 0.6399573, 0.23949516, 0.40301323, 0.8120787, 0.7841011,
    0.81297886, 0.70860887, 0.83292997, 0.036153078, 0.8052306, 0.05287373, 0.47609973,
    0.06610119, 0.1584791, 0.7400713, 0.045363903, 0.6977229, 0.7848017, 0.1330074,
    0.34231722, 0.12801087, 0.86731577, 0.3698523, 0.69349194, 0.97837555, 0.50665164,
    0.9837682, 0.99545264, 0.100491285, 0.82410944, 0.83790076, 0.8688601, 0.53486156,
    0.941082, 0.5920453, 0.79030395, 0.6920668, 0.7650684, 0.80544233, 0.8961737,
    0.79785836, 0.14100564, 0.35522497, 0.30713892, 0.18031168, 0.43362856, 0.016871095,
    0.56193995, 0.16531467, 0.22368371, 0.07928014, 0.003014803, 0.62796366, 0.76754165,
    0.41713, 0.37669563, 0.9529315, 0.34004676, 0.6092963, 0.3046757, 0.39615,
    0.24214983, 0.1650958, 0.061029434, 0.073996186, 0.057451606, 0.9826788, 0.987764,
    0.7065077, 0.9449332, 0.0418967, 0.10487056, 0.26843, 0.034175277, 0.81717086,
    0.1990552, 0.57412755, 0.05184543, 0.7586813, 0.7620821, 0.47232378, 0.78270936,
    0.45437014, 0.8479538, 0.09786272, 0.16876411, 0.46419013, 0.4561093, 0.48527944,
    0.6852114, 0.6509049, 0.072396874, 0.7788888, 0.19394803, 0.8333278, 0.5905111,
    0.8777, 0.14625561, 0.23645341, 0.23070085, 0.051487327, 0.05455792, 0.8493929,
    0.117322326, 0.8930719, 0.7717757, 0.22304487, 0.8634026, 0.52528596, 0.6994902,
    0.5021601, 0.3581773, 0.23868799, 0.4292748, 0.35127354, 0.36433637, 0.52105606,
    0.54196596, 0.6920922, 0.6005902, 0.3225528, 0.46457756, 0.9408618, 0.98212457,
    0.3598963, 0.9877881, 0.8807694, 0.99792075, 0.39707792, 0.9988047, 0.8472737,
    0.5996567, 0.7396703, 0.3473184, 0.3739065, 0.5479305, 0.27445686, 0.013775945,
    0.5311582, 0.6960057, 0.68597007, 0.45499122, 0.9454211, 0.36100876, 0.6663765,
    0.51741254, 0.93037975, 0.99563, 0.4777242, 0.7859727, 0.66935754, 0.6924659,
    0.15933561, 0.8039973, 0.120592594, 0.9558419, 0.59797347, 0.98089385, 0.2544949,
    0.54138243, 0.9694904, 0.9107065, 0.88294303, 0.08383775, 0.25321734, 0.9328828,
    0.71008575, 0.48014355, 0.8240315, 0.63994515, 0.24035895, 0.29259944, 0.19952428,
    0.10301936, 0.6840236, 0.2197541, 0.77775776, 0.7955445, 0.93995714, 0.7468102,
    0.22736764, 0.34684026, 0.32507694, 0.12116349, 0.9949782, 0.79268885, 0.7657956,
    0.6935377, 0.7885792, 0.18864012, 0.3491683, 0.39273798, 0.28253508, 0.60611117,
    0.4046234, 0.65809596, 0.38392782, 0.6724322, 0.77302146,
], dtype=np.float32)


_L = 16
_NCHUNK = N_LINES // _L


def _make_sc_kernel():
    info = plsc.get_sparse_core_info()
    nc, ns = info.num_cores, info.num_subcores
    nw = nc * ns
    rows_per_w = BATCH_SIZE // nw
    mesh = plsc.VectorSubcoreMesh(core_axis_name="c", subcore_axis_name="s")

    @functools.partial(
        pl.kernel,
        mesh=mesh,
        out_type=jax.ShapeDtypeStruct((BATCH_SIZE, N_LINES), jnp.float32),
        scratch_types=[
            pltpu.VMEM((N_LINES,), jnp.float32),
            pltpu.VMEM((N_LINES,), jnp.float32),
            pltpu.VMEM((rows_per_w, N_LINES), jnp.float32),
        ],
    )
    def k(logits_hbm, thr_hbm, out_hbm, x_v, t_v, rows_v):
        wid = lax.axis_index("s") * nc + lax.axis_index("c")
        pltpu.sync_copy(logits_hbm, x_v)
        pltpu.sync_copy(thr_hbm, t_v)
        acc = jnp.zeros((_L,), jnp.float32)
        for i in range(_NCHUNK):
            x = x_v[pl.ds(i * _L, _L)]
            p = 1.0 / (1.0 + jnp.exp(-5.0 * x))
            x_v[pl.ds(i * _L, _L)] = p
            acc = acc + p
        lane = lax.iota(jnp.int32, _L)
        dnums = lax.GatherDimensionNumbers(
            offset_dims=(), collapsed_slice_dims=(0,), start_index_map=(0,))
        for sh in (8, 4, 2, 1):
            perm = lax.gather(acc, (lane ^ sh)[:, None], dnums, slice_sizes=(1,),
                              mode=lax.GatherScatterMode.PROMISE_IN_BOUNDS)
            acc = acc + perm
        xbar = acc * (1.0 / N_LINES)
        cond = xbar > 0.125
        r = 0.125 / xbar
        beta = 0.875 / (1.0 - xbar)
        a = jnp.where(cond, r, beta)
        b = jnp.where(cond, 0.0, 1.0 - beta)
        for i in range(_NCHUNK):
            p = x_v[pl.ds(i * _L, _L)]
            t = t_v[pl.ds(i * _L, _L)]
            sm = 1.0 / (1.0 + jnp.exp(-12.0 * (a * p + b - t)))
            for j in range(rows_per_w):
                rows_v[j, pl.ds(i * _L, _L)] = sm
        pltpu.sync_copy(rows_v, out_hbm.at[pl.ds(wid * rows_per_w, rows_per_w)])

    return k


_sc_kernel = _make_sc_kernel()


def kernel(logits):
    thr = jnp.asarray(_THRESHOLDS)
    return _sc_kernel(logits, thr)


# speedup vs baseline: 1.0952x; 1.0952x over previous
import functools

import jax
import jax.numpy as jnp
import numpy as np
from jax import lax
from jax.experimental import pallas as pl
from jax.experimental.pallas import tpu as pltpu
from jax.experimental.pallas import tpu_sc as plsc

N_LINES = 208
BATCH_SIZE = 128

_THRESHOLDS = np.array([
    0.4386325, 0.5337529, 0.44591832, 0.43839633, 0.8973628, 0.558946, 0.9838855,
    0.84199095, 0.38577962, 0.6399573, 0.23949516, 0.40301323, 0.8120787, 0.7841011,
    0.81297886, 0.70860887, 0.83292997, 0.036153078, 0.8052306, 0.05287373, 0.47609973,
    0.06610119, 0.1584791, 0.7400713, 0.045363903, 0.6977229, 0.7848017, 0.1330074,
    0.34231722, 0.12801087, 0.86731577, 0.3698523, 0.69349194, 0.97837555, 0.50665164,
    0.9837682, 0.99545264, 0.100491285, 0.82410944, 0.83790076, 0.8688601, 0.53486156,
    0.941082, 0.5920453, 0.79030395, 0.6920668, 0.7650684, 0.80544233, 0.8961737,
    0.79785836, 0.14100564, 0.35522497, 0.30713892, 0.18031168, 0.43362856, 0.016871095,
    0.56193995, 0.16531467, 0.22368371, 0.07928014, 0.003014803, 0.62796366, 0.76754165,
    0.41713, 0.37669563, 0.9529315, 0.34004676, 0.6092963, 0.3046757, 0.39615,
    0.24214983, 0.1650958, 0.061029434, 0.073996186, 0.057451606, 0.9826788, 0.987764,
    0.7065077, 0.9449332, 0.0418967, 0.10487056, 0.26843, 0.034175277, 0.81717086,
    0.1990552, 0.57412755, 0.05184543, 0.7586813, 0.7620821, 0.47232378, 0.78270936,
    0.45437014, 0.8479538, 0.09786272, 0.16876411, 0.46419013, 0.4561093, 0.48527944,
    0.6852114, 0.6509049, 0.072396874, 0.7788888, 0.19394803, 0.8333278, 0.5905111,
    0.8777, 0.14625561, 0.23645341, 0.23070085, 0.051487327, 0.05455792, 0.8493929,
    0.117322326, 0.8930719, 0.7717757, 0.22304487, 0.8634026, 0.52528596, 0.6994902,
    0.5021601, 0.3581773, 0.23868799, 0.4292748, 0.35127354, 0.36433637, 0.52105606,
    0.54196596, 0.6920922, 0.6005902, 0.3225528, 0.46457756, 0.9408618, 0.98212457,
    0.3598963, 0.9877881, 0.8807694, 0.99792075, 0.39707792, 0.9988047, 0.8472737,
    0.5996567, 0.7396703, 0.3473184, 0.3739065, 0.5479305, 0.27445686, 0.013775945,
    0.5311582, 0.6960057, 0.68597007, 0.45499122, 0.9454211, 0.36100876, 0.6663765,
    0.51741254, 0.93037975, 0.99563, 0.4777242, 0.7859727, 0.66935754, 0.6924659,
    0.15933561, 0.8039973, 0.120592594, 0.9558419, 0.59797347, 0.98089385, 0.2544949,
    0.54138243, 0.9694904, 0.9107065, 0.88294303, 0.08383775, 0.25321734, 0.9328828,
    0.71008575, 0.48014355, 0.8240315, 0.63994515, 0.24035895, 0.29259944, 0.19952428,
    0.10301936, 0.6840236, 0.2197541, 0.77775776, 0.7955445, 0.93995714, 0.7468102,
    0.22736764, 0.34684026, 0.32507694, 0.12116349, 0.9949782, 0.79268885, 0.7657956,
    0.6935377, 0.7885792, 0.18864012, 0.3491683, 0.39273798, 0.28253508, 0.60611117,
    0.4046234, 0.65809596, 0.38392782, 0.6724322, 0.77302146,
], dtype=np.float32)


_L = 16
_NCHUNK = N_LINES // _L


def _make_sc_kernel(num_cores):
    info = plsc.get_sparse_core_info()
    nc, ns = num_cores, info.num_subcores
    nw = nc * ns
    rows_per_w = BATCH_SIZE // nw
    mesh = plsc.VectorSubcoreMesh(
        core_axis_name="c", subcore_axis_name="s", num_cores=nc)

    @functools.partial(
        pl.kernel,
        mesh=mesh,
        out_type=jax.ShapeDtypeStruct((BATCH_SIZE, N_LINES), jnp.float32),
        scratch_types=[
            pltpu.VMEM((N_LINES,), jnp.float32),
            pltpu.VMEM((N_LINES,), jnp.float32),
            pltpu.VMEM((rows_per_w, N_LINES), jnp.float32),
            pltpu.SemaphoreType.DMA,
            pltpu.SemaphoreType.DMA,
        ],
    )
    def k(logits_hbm, thr_hbm, out_hbm, x_v, t_v, rows_v, xsem, tsem):
        wid = lax.axis_index("s") * nc + lax.axis_index("c")
        xcp = pltpu.async_copy(logits_hbm, x_v, xsem)
        tcp = pltpu.async_copy(thr_hbm, t_v, tsem)
        xcp.wait()
        acc = jnp.zeros((_L,), jnp.float32)
        for i in range(_NCHUNK):
            x = x_v[pl.ds(i * _L, _L)]
            p = 1.0 / (1.0 + jnp.exp(-5.0 * x))
            x_v[pl.ds(i * _L, _L)] = p
            acc = acc + p
        lane = lax.iota(jnp.int32, _L)
        dnums = lax.GatherDimensionNumbers(
            offset_dims=(), collapsed_slice_dims=(0,), start_index_map=(0,))
        for sh in (8, 4, 2, 1):
            perm = lax.gather(acc, (lane ^ sh)[:, None], dnums, slice_sizes=(1,),
                              mode=lax.GatherScatterMode.PROMISE_IN_BOUNDS)
            acc = acc + perm
        xbar = acc * (1.0 / N_LINES)
        cond = xbar > 0.125
        r = 0.125 / xbar
        beta = 0.875 / (1.0 - xbar)
        a = jnp.where(cond, r, beta)
        b = jnp.where(cond, 0.0, 1.0 - beta)
        tcp.wait()
        for i in range(_NCHUNK):
            p = x_v[pl.ds(i * _L, _L)]
            t = t_v[pl.ds(i * _L, _L)]
            sm = 1.0 / (1.0 + jnp.exp(-12.0 * (a * p + b - t)))
            for j in range(rows_per_w):
                rows_v[j, pl.ds(i * _L, _L)] = sm
        pltpu.sync_copy(rows_v, out_hbm.at[pl.ds(wid * rows_per_w, rows_per_w)])

    return k


_sc_kernel = _make_sc_kernel(num_cores=1)


def kernel(logits):
    thr = jnp.asarray(_THRESHOLDS)
    return _sc_kernel(logits, thr)


# speedup vs baseline: 7.5734x; 6.9148x over previous
import jax
import jax.numpy as jnp
import numpy as np
from jax.experimental import pallas as pl

N_LINES = 208
BATCH_SIZE = 128

_THRESHOLDS = np.array([
    0.4386325, 0.5337529, 0.44591832, 0.43839633, 0.8973628, 0.558946, 0.9838855,
    0.84199095, 0.38577962, 0.6399573, 0.23949516, 0.40301323, 0.8120787, 0.7841011,
    0.81297886, 0.70860887, 0.83292997, 0.036153078, 0.8052306, 0.05287373, 0.47609973,
    0.06610119, 0.1584791, 0.7400713, 0.045363903, 0.6977229, 0.7848017, 0.1330074,
    0.34231722, 0.12801087, 0.86731577, 0.3698523, 0.69349194, 0.97837555, 0.50665164,
    0.9837682, 0.99545264, 0.100491285, 0.82410944, 0.83790076, 0.8688601, 0.53486156,
    0.941082, 0.5920453, 0.79030395, 0.6920668, 0.7650684, 0.80544233, 0.8961737,
    0.79785836, 0.14100564, 0.35522497, 0.30713892, 0.18031168, 0.43362856, 0.016871095,
    0.56193995, 0.16531467, 0.22368371, 0.07928014, 0.003014803, 0.62796366, 0.76754165,
    0.41713, 0.37669563, 0.9529315, 0.34004676, 0.6092963, 0.3046757, 0.39615,
    0.24214983, 0.1650958, 0.061029434, 0.073996186, 0.057451606, 0.9826788, 0.987764,
    0.7065077, 0.9449332, 0.0418967, 0.10487056, 0.26843, 0.034175277, 0.81717086,
    0.1990552, 0.57412755, 0.05184543, 0.7586813, 0.7620821, 0.47232378, 0.78270936,
    0.45437014, 0.8479538, 0.09786272, 0.16876411, 0.46419013, 0.4561093, 0.48527944,
    0.6852114, 0.6509049, 0.072396874, 0.7788888, 0.19394803, 0.8333278, 0.5905111,
    0.8777, 0.14625561, 0.23645341, 0.23070085, 0.051487327, 0.05455792, 0.8493929,
    0.117322326, 0.8930719, 0.7717757, 0.22304487, 0.8634026, 0.52528596, 0.6994902,
    0.5021601, 0.3581773, 0.23868799, 0.4292748, 0.35127354, 0.36433637, 0.52105606,
    0.54196596, 0.6920922, 0.6005902, 0.3225528, 0.46457756, 0.9408618, 0.98212457,
    0.3598963, 0.9877881, 0.8807694, 0.99792075, 0.39707792, 0.9988047, 0.8472737,
    0.5996567, 0.7396703, 0.3473184, 0.3739065, 0.5479305, 0.27445686, 0.013775945,
    0.5311582, 0.6960057, 0.68597007, 0.45499122, 0.9454211, 0.36100876, 0.6663765,
    0.51741254, 0.93037975, 0.99563, 0.4777242, 0.7859727, 0.66935754, 0.6924659,
    0.15933561, 0.8039973, 0.120592594, 0.9558419, 0.59797347, 0.98089385, 0.2544949,
    0.54138243, 0.9694904, 0.9107065, 0.88294303, 0.08383775, 0.25321734, 0.9328828,
    0.71008575, 0.48014355, 0.8240315, 0.63994515, 0.24035895, 0.29259944, 0.19952428,
    0.10301936, 0.6840236, 0.2197541, 0.77775776, 0.7955445, 0.93995714, 0.7468102,
    0.22736764, 0.34684026, 0.32507694, 0.12116349, 0.9949782, 0.79268885, 0.7657956,
    0.6935377, 0.7885792, 0.18864012, 0.3491683, 0.39273798, 0.28253508, 0.60611117,
    0.4046234, 0.65809596, 0.38392782, 0.6724322, 0.77302146,
], dtype=np.float32)


def _body(x_ref, t_ref, o_ref):
    x = x_ref[...]
    p = jax.nn.sigmoid(5.0 * x)
    xbar = jnp.mean(p)
    r = 0.125 / xbar
    beta = 0.875 / (1.0 - xbar)
    scaled = jnp.where(xbar > 0.125, p * r, 1.0 - (1.0 - p) * beta)
    s = jax.nn.sigmoid(12.0 * (scaled - t_ref[...]))
    o_ref[...] = jnp.broadcast_to(s, (BATCH_SIZE, N_LINES))


def kernel(logits):
    x = logits.reshape(1, N_LINES)
    thr = jnp.asarray(_THRESHOLDS).reshape(1, N_LINES)
    return pl.pallas_call(
        _body,
        out_shape=jax.ShapeDtypeStruct((BATCH_SIZE, N_LINES), jnp.float32),
    )(x, thr)


# speedup vs baseline: 14.3855x; 1.8995x over previous
import jax
import jax.numpy as jnp
import numpy as np
from jax.experimental import pallas as pl

N_LINES = 208
BATCH_SIZE = 128

_THRESHOLDS = np.array([
    0.4386325, 0.5337529, 0.44591832, 0.43839633, 0.8973628, 0.558946, 0.9838855,
    0.84199095, 0.38577962, 0.6399573, 0.23949516, 0.40301323, 0.8120787, 0.7841011,
    0.81297886, 0.70860887, 0.83292997, 0.036153078, 0.8052306, 0.05287373, 0.47609973,
    0.06610119, 0.1584791, 0.7400713, 0.045363903, 0.6977229, 0.7848017, 0.1330074,
    0.34231722, 0.12801087, 0.86731577, 0.3698523, 0.69349194, 0.97837555, 0.50665164,
    0.9837682, 0.99545264, 0.100491285, 0.82410944, 0.83790076, 0.8688601, 0.53486156,
    0.941082, 0.5920453, 0.79030395, 0.6920668, 0.7650684, 0.80544233, 0.8961737,
    0.79785836, 0.14100564, 0.35522497, 0.30713892, 0.18031168, 0.43362856, 0.016871095,
    0.56193995, 0.16531467, 0.22368371, 0.07928014, 0.003014803, 0.62796366, 0.76754165,
    0.41713, 0.37669563, 0.9529315, 0.34004676, 0.6092963, 0.3046757, 0.39615,
    0.24214983, 0.1650958, 0.061029434, 0.073996186, 0.057451606, 0.9826788, 0.987764,
    0.7065077, 0.9449332, 0.0418967, 0.10487056, 0.26843, 0.034175277, 0.81717086,
    0.1990552, 0.57412755, 0.05184543, 0.7586813, 0.7620821, 0.47232378, 0.78270936,
    0.45437014, 0.8479538, 0.09786272, 0.16876411, 0.46419013, 0.4561093, 0.48527944,
    0.6852114, 0.6509049, 0.072396874, 0.7788888, 0.19394803, 0.8333278, 0.5905111,
    0.8777, 0.14625561, 0.23645341, 0.23070085, 0.051487327, 0.05455792, 0.8493929,
    0.117322326, 0.8930719, 0.7717757, 0.22304487, 0.8634026, 0.52528596, 0.6994902,
    0.5021601, 0.3581773, 0.23868799, 0.4292748, 0.35127354, 0.36433637, 0.52105606,
    0.54196596, 0.6920922, 0.6005902, 0.3225528, 0.46457756, 0.9408618, 0.98212457,
    0.3598963, 0.9877881, 0.8807694, 0.99792075, 0.39707792, 0.9988047, 0.8472737,
    0.5996567, 0.7396703, 0.3473184, 0.3739065, 0.5479305, 0.27445686, 0.013775945,
    0.5311582, 0.6960057, 0.68597007, 0.45499122, 0.9454211, 0.36100876, 0.6663765,
    0.51741254, 0.93037975, 0.99563, 0.4777242, 0.7859727, 0.66935754, 0.6924659,
    0.15933561, 0.8039973, 0.120592594, 0.9558419, 0.59797347, 0.98089385, 0.2544949,
    0.54138243, 0.9694904, 0.9107065, 0.88294303, 0.08383775, 0.25321734, 0.9328828,
    0.71008575, 0.48014355, 0.8240315, 0.63994515, 0.24035895, 0.29259944, 0.19952428,
    0.10301936, 0.6840236, 0.2197541, 0.77775776, 0.7955445, 0.93995714, 0.7468102,
    0.22736764, 0.34684026, 0.32507694, 0.12116349, 0.9949782, 0.79268885, 0.7657956,
    0.6935377, 0.7885792, 0.18864012, 0.3491683, 0.39273798, 0.28253508, 0.60611117,
    0.4046234, 0.65809596, 0.38392782, 0.6724322, 0.77302146,
], dtype=np.float32)


def _body(x_ref, t_ref, o_ref):
    x = x_ref[...]
    p = jax.nn.sigmoid(5.0 * x)
    xbar = jnp.mean(p)
    r = 0.125 / xbar
    beta = 0.875 / (1.0 - xbar)
    scaled = jnp.where(xbar > 0.125, p * r, 1.0 - (1.0 - p) * beta)
    s = jax.nn.sigmoid(12.0 * (scaled - t_ref[...]))
    o_ref[...] = jax.lax.broadcast_in_dim(
        s.reshape(N_LINES), (N_LINES, BATCH_SIZE), (0,))


def kernel(logits):
    x = logits.reshape(1, N_LINES)
    thr = jnp.asarray(_THRESHOLDS).reshape(1, N_LINES)
    out = pl.pallas_call(
        _body,
        out_shape=jax.ShapeDtypeStruct((N_LINES, BATCH_SIZE), jnp.float32),
    )(x, thr)
    return out.T


# speedup vs baseline: 14.5788x; 1.0134x over previous
import jax
import jax.numpy as jnp
import numpy as np
from jax.experimental import pallas as pl
from jax.experimental.pallas import tpu as pltpu

N_LINES = 208
BATCH_SIZE = 128

_THRESHOLDS = np.array([
    0.4386325, 0.5337529, 0.44591832, 0.43839633, 0.8973628, 0.558946, 0.9838855,
    0.84199095, 0.38577962, 0.6399573, 0.23949516, 0.40301323, 0.8120787, 0.7841011,
    0.81297886, 0.70860887, 0.83292997, 0.036153078, 0.8052306, 0.05287373, 0.47609973,
    0.06610119, 0.1584791, 0.7400713, 0.045363903, 0.6977229, 0.7848017, 0.1330074,
    0.34231722, 0.12801087, 0.86731577, 0.3698523, 0.69349194, 0.97837555, 0.50665164,
    0.9837682, 0.99545264, 0.100491285, 0.82410944, 0.83790076, 0.8688601, 0.53486156,
    0.941082, 0.5920453, 0.79030395, 0.6920668, 0.7650684, 0.80544233, 0.8961737,
    0.79785836, 0.14100564, 0.35522497, 0.30713892, 0.18031168, 0.43362856, 0.016871095,
    0.56193995, 0.16531467, 0.22368371, 0.07928014, 0.003014803, 0.62796366, 0.76754165,
    0.41713, 0.37669563, 0.9529315, 0.34004676, 0.6092963, 0.3046757, 0.39615,
    0.24214983, 0.1650958, 0.061029434, 0.073996186, 0.057451606, 0.9826788, 0.987764,
    0.7065077, 0.9449332, 0.0418967, 0.10487056, 0.26843, 0.034175277, 0.81717086,
    0.1990552, 0.57412755, 0.05184543, 0.7586813, 0.7620821, 0.47232378, 0.78270936,
    0.45437014, 0.8479538, 0.09786272, 0.16876411, 0.46419013, 0.4561093, 0.48527944,
    0.6852114, 0.6509049, 0.072396874, 0.7788888, 0.19394803, 0.8333278, 0.5905111,
    0.8777, 0.14625561, 0.23645341, 0.23070085, 0.051487327, 0.05455792, 0.8493929,
    0.117322326, 0.8930719, 0.7717757, 0.22304487, 0.8634026, 0.52528596, 0.6994902,
    0.5021601, 0.3581773, 0.23868799, 0.4292748, 0.35127354, 0.36433637, 0.52105606,
    0.54196596, 0.6920922, 0.6005902, 0.3225528, 0.46457756, 0.9408618, 0.98212457,
    0.3598963, 0.9877881, 0.8807694, 0.99792075, 0.39707792, 0.9988047, 0.8472737,
    0.5996567, 0.7396703, 0.3473184, 0.3739065, 0.5479305, 0.27445686, 0.013775945,
    0.5311582, 0.6960057, 0.68597007, 0.45499122, 0.9454211, 0.36100876, 0.6663765,
    0.51741254, 0.93037975, 0.99563, 0.4777242, 0.7859727, 0.66935754, 0.6924659,
    0.15933561, 0.8039973, 0.120592594, 0.9558419, 0.59797347, 0.98089385, 0.2544949,
    0.54138243, 0.9694904, 0.9107065, 0.88294303, 0.08383775, 0.25321734, 0.9328828,
    0.71008575, 0.48014355, 0.8240315, 0.63994515, 0.24035895, 0.29259944, 0.19952428,
    0.10301936, 0.6840236, 0.2197541, 0.77775776, 0.7955445, 0.93995714, 0.7468102,
    0.22736764, 0.34684026, 0.32507694, 0.12116349, 0.9949782, 0.79268885, 0.7657956,
    0.6935377, 0.7885792, 0.18864012, 0.3491683, 0.39273798, 0.28253508, 0.60611117,
    0.4046234, 0.65809596, 0.38392782, 0.6724322, 0.77302146,
], dtype=np.float32)


_HALF = N_LINES // 2


def _body(x_ref, t_ref, o_ref, buf, sem0, sem1):
    x = x_ref[...]
    p = jax.nn.sigmoid(5.0 * x)
    xbar = jnp.mean(p)
    r = 0.125 / xbar
    beta = 0.875 / (1.0 - xbar)
    scaled = jnp.where(xbar > 0.125, p * r, 1.0 - (1.0 - p) * beta)
    s = jax.nn.sigmoid(12.0 * (scaled - t_ref[...])).reshape(N_LINES)
    buf[0:_HALF, :] = jax.lax.broadcast_in_dim(
        s[0:_HALF], (_HALF, BATCH_SIZE), (0,))
    cp0 = pltpu.make_async_copy(buf.at[0:_HALF], o_ref.at[0:_HALF], sem0)
    cp0.start()
    buf[_HALF:N_LINES, :] = jax.lax.broadcast_in_dim(
        s[_HALF:N_LINES], (_HALF, BATCH_SIZE), (0,))
    cp1 = pltpu.make_async_copy(
        buf.at[_HALF:N_LINES], o_ref.at[_HALF:N_LINES], sem1)
    cp1.start()
    cp0.wait()
    cp1.wait()


def kernel(logits):
    x = logits.reshape(1, N_LINES)
    thr = jnp.asarray(_THRESHOLDS).reshape(1, N_LINES)
    out = pl.pallas_call(
        _body,
        out_shape=jax.ShapeDtypeStruct((N_LINES, BATCH_SIZE), jnp.float32),
        out_specs=pl.BlockSpec(memory_space=pltpu.MemorySpace.HBM),
        scratch_shapes=[
            pltpu.VMEM((N_LINES, BATCH_SIZE), jnp.float32),
            pltpu.SemaphoreType.DMA,
            pltpu.SemaphoreType.DMA,
        ],
    )(x, thr)
    return out.T


# speedup vs baseline: 15.0762x; 1.0341x over previous
import jax
import jax.numpy as jnp
import numpy as np
from jax.experimental import pallas as pl
from jax.experimental.pallas import tpu as pltpu

N_LINES = 208
BATCH_SIZE = 128

_THRESHOLDS = np.array([
    0.4386325, 0.5337529, 0.44591832, 0.43839633, 0.8973628, 0.558946, 0.9838855,
    0.84199095, 0.38577962, 0.6399573, 0.23949516, 0.40301323, 0.8120787, 0.7841011,
    0.81297886, 0.70860887, 0.83292997, 0.036153078, 0.8052306, 0.05287373, 0.47609973,
    0.06610119, 0.1584791, 0.7400713, 0.045363903, 0.6977229, 0.7848017, 0.1330074,
    0.34231722, 0.12801087, 0.86731577, 0.3698523, 0.69349194, 0.97837555, 0.50665164,
    0.9837682, 0.99545264, 0.100491285, 0.82410944, 0.83790076, 0.8688601, 0.53486156,
    0.941082, 0.5920453, 0.79030395, 0.6920668, 0.7650684, 0.80544233, 0.8961737,
    0.79785836, 0.14100564, 0.35522497, 0.30713892, 0.18031168, 0.43362856, 0.016871095,
    0.56193995, 0.16531467, 0.22368371, 0.07928014, 0.003014803, 0.62796366, 0.76754165,
    0.41713, 0.37669563, 0.9529315, 0.34004676, 0.6092963, 0.3046757, 0.39615,
    0.24214983, 0.1650958, 0.061029434, 0.073996186, 0.057451606, 0.9826788, 0.987764,
    0.7065077, 0.9449332, 0.0418967, 0.10487056, 0.26843, 0.034175277, 0.81717086,
    0.1990552, 0.57412755, 0.05184543, 0.7586813, 0.7620821, 0.47232378, 0.78270936,
    0.45437014, 0.8479538, 0.09786272, 0.16876411, 0.46419013, 0.4561093, 0.48527944,
    0.6852114, 0.6509049, 0.072396874, 0.7788888, 0.19394803, 0.8333278, 0.5905111,
    0.8777, 0.14625561, 0.23645341, 0.23070085, 0.051487327, 0.05455792, 0.8493929,
    0.117322326, 0.8930719, 0.7717757, 0.22304487, 0.8634026, 0.52528596, 0.6994902,
    0.5021601, 0.3581773, 0.23868799, 0.4292748, 0.35127354, 0.36433637, 0.52105606,
    0.54196596, 0.6920922, 0.6005902, 0.3225528, 0.46457756, 0.9408618, 0.98212457,
    0.3598963, 0.9877881, 0.8807694, 0.99792075, 0.39707792, 0.9988047, 0.8472737,
    0.5996567, 0.7396703, 0.3473184, 0.3739065, 0.5479305, 0.27445686, 0.013775945,
    0.5311582, 0.6960057, 0.68597007, 0.45499122, 0.9454211, 0.36100876, 0.6663765,
    0.51741254, 0.93037975, 0.99563, 0.4777242, 0.7859727, 0.66935754, 0.6924659,
    0.15933561, 0.8039973, 0.120592594, 0.9558419, 0.59797347, 0.98089385, 0.2544949,
    0.54138243, 0.9694904, 0.9107065, 0.88294303, 0.08383775, 0.25321734, 0.9328828,
    0.71008575, 0.48014355, 0.8240315, 0.63994515, 0.24035895, 0.29259944, 0.19952428,
    0.10301936, 0.6840236, 0.2197541, 0.77775776, 0.7955445, 0.93995714, 0.7468102,
    0.22736764, 0.34684026, 0.32507694, 0.12116349, 0.9949782, 0.79268885, 0.7657956,
    0.6935377, 0.7885792, 0.18864012, 0.3491683, 0.39273798, 0.28253508, 0.60611117,
    0.4046234, 0.65809596, 0.38392782, 0.6724322, 0.77302146,
], dtype=np.float32)


_HALF = N_LINES // 2


def _body(x_hbm, t_hbm, o_ref, xb, tb, buf, semx, semt, sem0, sem1):
    cpx = pltpu.make_async_copy(x_hbm, xb, semx)
    cpx.start()
    cpt = pltpu.make_async_copy(t_hbm, tb, semt)
    cpt.start()
    cpx.wait()
    x = xb[...]
    p = jax.nn.sigmoid(5.0 * x)
    xbar = jnp.mean(p)
    r = 0.125 / xbar
    beta = 0.875 / (1.0 - xbar)
    scaled = jnp.where(xbar > 0.125, p * r, 1.0 - (1.0 - p) * beta)
    cpt.wait()
    s = jax.nn.sigmoid(12.0 * (scaled - tb[...])).reshape(N_LINES)
    buf[0:_HALF, :] = jax.lax.broadcast_in_dim(
        s[0:_HALF], (_HALF, BATCH_SIZE), (0,))
    cp0 = pltpu.make_async_copy(buf.at[0:_HALF], o_ref.at[0:_HALF], sem0)
    cp0.start()
    buf[_HALF:N_LINES, :] = jax.lax.broadcast_in_dim(
        s[_HALF:N_LINES], (_HALF, BATCH_SIZE), (0,))
    cp1 = pltpu.make_async_copy(
        buf.at[_HALF:N_LINES], o_ref.at[_HALF:N_LINES], sem1)
    cp1.start()
    cp0.wait()
    cp1.wait()


def kernel(logits):
    x = logits.reshape(1, N_LINES)
    thr = jnp.asarray(_THRESHOLDS).reshape(1, N_LINES)
    out = pl.pallas_call(
        _body,
        out_shape=jax.ShapeDtypeStruct((N_LINES, BATCH_SIZE), jnp.float32),
        in_specs=[pl.BlockSpec(memory_space=pltpu.MemorySpace.HBM),
                  pl.BlockSpec(memory_space=pltpu.MemorySpace.HBM)],
        out_specs=pl.BlockSpec(memory_space=pltpu.MemorySpace.HBM),
        scratch_shapes=[
            pltpu.VMEM((1, N_LINES), jnp.float32),
            pltpu.VMEM((1, N_LINES), jnp.float32),
            pltpu.VMEM((N_LINES, BATCH_SIZE), jnp.float32),
            pltpu.SemaphoreType.DMA,
            pltpu.SemaphoreType.DMA,
            pltpu.SemaphoreType.DMA,
            pltpu.SemaphoreType.DMA,
        ],
    )(x, thr)
    return out.T
